# baseline (speedup 1.0000x reference)
import functools

import jax
import jax.numpy as jnp
from jax import lax
from jax.experimental import pallas as pl
from jax.experimental.pallas import tpu as pltpu
from jax.experimental.pallas import tpu_sc as plsc

EMBEDDING_DIM = 64

_NUM_CORES = 2
_NUM_SUBCORES = 16
_NUM_WORKERS = _NUM_CORES * _NUM_SUBCORES

_CHUNK = 1600


@functools.partial(jax.jit, static_argnames=("num_indices",))
def _embedding_gather(weight, flat_ids, *, num_indices):
    b_per_w = num_indices // _NUM_WORKERS
    n_chunks = b_per_w // _CHUNK
    mesh = plsc.VectorSubcoreMesh(core_axis_name="c", subcore_axis_name="s")

    @functools.partial(
        pl.kernel,
        mesh=mesh,
        compiler_params=pltpu.CompilerParams(use_tc_tiling_on_sc=False),
        out_type=jax.ShapeDtypeStruct((num_indices, EMBEDDING_DIM), jnp.float32),
        scratch_types=[
            pltpu.VMEM((_CHUNK,), jnp.int32),
            pltpu.VMEM((_CHUNK, EMBEDDING_DIM), jnp.float32),
            pltpu.SemaphoreType.DMA,
        ],
    )
    def gather_kernel(table_hbm, idx_hbm, out_hbm, idx_v, rows_v, sem):
        wid = lax.axis_index("s") * _NUM_CORES + lax.axis_index("c")
        base = wid * b_per_w

        def step(g, carry):
            off = base + g * _CHUNK
            pltpu.sync_copy(idx_hbm.at[pl.ds(off, _CHUNK)], idx_v)
            pltpu.async_copy(table_hbm.at[idx_v], rows_v, sem).wait()
            pltpu.sync_copy(rows_v, out_hbm.at[pl.ds(off, _CHUNK)])
            return carry

        lax.fori_loop(0, n_chunks, step, 0)

    return gather_kernel(weight, flat_ids)


def kernel(token_ids, weight):
    batch, seq = token_ids.shape
    flat = token_ids.reshape(-1).astype(jnp.int32)
    out = _embedding_gather(weight, flat, num_indices=batch * seq)
    return out.reshape(batch, seq, EMBEDDING_DIM)


# speedup vs baseline: 1.0056x; 1.0056x over previous
import functools

import jax
import jax.numpy as jnp
from jax import lax
from jax.experimental import pallas as pl
from jax.experimental.pallas import tpu as pltpu
from jax.experimental.pallas import tpu_sc as plsc

EMBEDDING_DIM = 64

_NUM_CORES = 2
_NUM_SUBCORES = 16
_NUM_WORKERS = _NUM_CORES * _NUM_SUBCORES

_CHUNK = 512
_NBUF = 3


@functools.partial(jax.jit, static_argnames=("num_indices",))
def _embedding_gather(weight, flat_ids, *, num_indices):
    b_per_w = num_indices // _NUM_WORKERS
    n_chunks = b_per_w // _CHUNK
    mesh = plsc.VectorSubcoreMesh(core_axis_name="c", subcore_axis_name="s")

    @functools.partial(
        pl.kernel,
        mesh=mesh,
        compiler_params=pltpu.CompilerParams(use_tc_tiling_on_sc=False),
        out_type=jax.ShapeDtypeStruct((num_indices, EMBEDDING_DIM), jnp.float32),
        scratch_types=[
            pltpu.VMEM((b_per_w,), jnp.int32),
            *[pltpu.VMEM((_CHUNK, EMBEDDING_DIM), jnp.float32) for _ in range(_NBUF)],
            *[pltpu.SemaphoreType.DMA for _ in range(2 * _NBUF)],
        ],
    )
    def gather_kernel(table_hbm, idx_hbm, out_hbm, idx_v, *bufs_and_sems):
        rows = bufs_and_sems[:_NBUF]
        gsem = bufs_and_sems[_NBUF : 2 * _NBUF]
        osem = bufs_and_sems[2 * _NBUF : 3 * _NBUF]

        wid = lax.axis_index("s") * _NUM_CORES + lax.axis_index("c")
        base = wid * b_per_w

        pltpu.sync_copy(idx_hbm.at[pl.ds(base, b_per_w)], idx_v)

        def start_gather(g):
            b = g % _NBUF
            return pltpu.async_copy(
                table_hbm.at[idx_v.at[pl.ds(g * _CHUNK, _CHUNK)]], rows[b], gsem[b]
            )

        def start_out(g):
            b = g % _NBUF
            return pltpu.async_copy(
                rows[b], out_hbm.at[pl.ds(base + g * _CHUNK, _CHUNK)], osem[b]
            )

        gathers = [None] * n_chunks
        outs = [None] * n_chunks
        gathers[0] = start_gather(0)
        if n_chunks > 1:
            gathers[1] = start_gather(1)
        for g in range(n_chunks):
            gathers[g].wait()
            outs[g] = start_out(g)
            g2 = g + 2
            if g2 < n_chunks:
                if g2 >= _NBUF:
                    outs[g2 - _NBUF].wait()
                gathers[g2] = start_gather(g2)
        for g in range(max(0, n_chunks - _NBUF), n_chunks):
            outs[g].wait()

    return gather_kernel(weight, flat_ids)


def kernel(token_ids, weight):
    batch, seq = token_ids.shape
    flat = token_ids.reshape(-1).astype(jnp.int32)
    out = _embedding_gather(weight, flat, num_indices=batch * seq)
    return out.reshape(batch, seq, EMBEDDING_DIM)


# speedup vs baseline: 1.0066x; 1.0010x over previous
import functools

import jax
import jax.numpy as jnp
from jax import lax
from jax.experimental import pallas as pl
from jax.experimental.pallas import tpu as pltpu
from jax.experimental.pallas import tpu_sc as plsc

EMBEDDING_DIM = 64

_NUM_CORES = 2
_NUM_SUBCORES = 16
_NUM_WORKERS = _NUM_CORES * _NUM_SUBCORES

_CHUNK = 512
_NBUF = 3


@functools.partial(jax.jit, static_argnames=("num_indices",))
def _embedding_gather(weight, flat_ids, *, num_indices):
    b_per_w = num_indices // _NUM_WORKERS
    n_chunks = b_per_w // _CHUNK
    mesh = plsc.VectorSubcoreMesh(core_axis_name="c", subcore_axis_name="s")

    @functools.partial(
        pl.kernel,
        mesh=mesh,
        compiler_params=pltpu.CompilerParams(
            use_tc_tiling_on_sc=False, skip_device_barrier=True
        ),
        out_type=jax.ShapeDtypeStruct((num_indices, EMBEDDING_DIM), jnp.float32),
        scratch_types=[
            pltpu.VMEM((b_per_w,), jnp.int32),
            *[pltpu.VMEM((_CHUNK, EMBEDDING_DIM), jnp.float32) for _ in range(_NBUF)],
            *[pltpu.SemaphoreType.DMA for _ in range(2 * _NBUF)],
        ],
    )
    def gather_kernel(table_hbm, idx_hbm, out_hbm, idx_v, *bufs_and_sems):
        rows = bufs_and_sems[:_NBUF]
        gsem = bufs_and_sems[_NBUF : 2 * _NBUF]
        osem = bufs_and_sems[2 * _NBUF : 3 * _NBUF]

        wid = lax.axis_index("s") * _NUM_CORES + lax.axis_index("c")
        base = wid * b_per_w

        pltpu.sync_copy(idx_hbm.at[pl.ds(base, b_per_w)], idx_v)

        def start_gather(g):
            b = g % _NBUF
            return pltpu.async_copy(
                table_hbm.at[idx_v.at[pl.ds(g * _CHUNK, _CHUNK)]], rows[b], gsem[b]
            )

        def start_out(g):
            b = g % _NBUF
            return pltpu.async_copy(
                rows[b], out_hbm.at[pl.ds(base + g * _CHUNK, _CHUNK)], osem[b]
            )

        gathers = [None] * n_chunks
        outs = [None] * n_chunks
        gathers[0] = start_gather(0)
        if n_chunks > 1:
            gathers[1] = start_gather(1)
        for g in range(n_chunks):
            gathers[g].wait()
            outs[g] = start_out(g)
            g2 = g + 2
            if g2 < n_chunks:
                if g2 >= _NBUF:
                    outs[g2 - _NBUF].wait()
                gathers[g2] = start_gather(g2)
        for g in range(max(0, n_chunks - _NBUF), n_chunks):
            outs[g].wait()

    return gather_kernel(weight, flat_ids)


def kernel(token_ids, weight):
    batch, seq = token_ids.shape
    flat = token_ids.reshape(-1).astype(jnp.int32)
    out = _embedding_gather(weight, flat, num_indices=batch * seq)
    return out.reshape(batch, seq, EMBEDDING_DIM)
